# Initial kernel scaffold; baseline (speedup 1.0000x reference)
import jax
import jax.numpy as jnp
from jax.experimental import pallas as pl


def kernel(holed_img, idx, dist, filled_idx, unfilled_idx):
    raise NotImplementedError("write your pallas kernel here")


# speedup vs baseline: 1.4296x; 1.4296x over previous
import jax
import jax.numpy as jnp
from jax import lax
from jax.experimental import pallas as pl
from jax.experimental.pallas import tpu as pltpu
from jax.experimental.pallas import tpu_sc as plsc

B, H, W = 32, 512, 512
HW = H * W
U = 131072
F_ = 131072
K = 8
NC, NS, L = 2, 16, 16
NW = NC * NS
UPW = U // NW
CH = 64
NCHUNK = UPW // CH
GRP = 8
FPW = F_ // NW
PR = 128
NPERM = FPW // PR
HWPW = HW // NW
NCOPY = HWPW // PR
CGRP = 8
DUMP = HW
_mesh = plsc.VectorSubcoreMesh(core_axis_name="c", subcore_axis_name="s")
_sc_params = pltpu.CompilerParams(use_tc_tiling_on_sc=False)


def _wid():
    return lax.axis_index("s") * NC + lax.axis_index("c")


def _permute_body(img_hbm, lin_hbm, imgp_hbm, lbuf0, lbuf1, prows,
                  lsem0, lsem1, gsem0, gsem1, psem0, psem1):
    wid = _wid()
    f0 = wid * FPW
    lbuf = (lbuf0, lbuf1)
    lsem = (lsem0, lsem1)
    gsem = (gsem0, gsem1)
    psem = (psem0, psem1)
    lin_d = [None, None]
    g_d = [None, None]
    s_d = [None, None]

    def lin_load(r, pb):
        lin_d[pb] = pltpu.async_copy(
            lin_hbm.at[pl.ds(f0 + r * PR, PR)], lbuf[pb], lsem[pb])

    def gather(pb):
        g_d[pb] = pltpu.async_copy(img_hbm.at[lbuf[pb]], prows.at[pb],
                                   gsem[pb])

    def store(r, pb):
        s_d[pb] = pltpu.async_copy(
            prows.at[pb], imgp_hbm.at[pl.ds(f0 + r * PR, PR), :], psem[pb])

    lin_load(0, 0)
    lin_load(1, 1)
    lin_d[0].wait()
    gather(0)
    for r in range(NPERM):
        pb = r & 1
        if r + 1 < NPERM:
            lin_d[pb ^ 1].wait()
            if s_d[pb ^ 1] is not None:
                s_d[pb ^ 1].wait()
            gather(pb ^ 1)
        g_d[pb].wait()
        store(r, pb)
        if r + 2 < NPERM:
            lin_load(r + 2, pb)
    s_d[0].wait()
    s_d[1].wait()


def _fill_body(imgp_hbm, idx_hbm, dist_hbm, ulin_hbm, win_hbm, img_hbm,
               out_hbm, ulbuf, winu, uidx0, uidx1,
               ib00, ib01, ib02, ib03, ib10, ib11, ib12, ib13,
               rows, dbuf, vals, sul0, sul1, crows, winc,
               cidx0, cidx1, cidx2, cidx3,
               asem0, asem1, lsem0, lsem1, isem0, isem1, rsem0, rsem1,
               dsem0, dsem1, ssem0, ssem1, clsem0, clsem1, clsem2, clsem3,
               csem0, csem1, csem2, csem3):
    wid = _wid()
    u0 = wid * UPW
    iota = lax.broadcasted_iota(jnp.int32, (L,), 0)
    uidx = (uidx0, uidx1)
    ib = ((ib00, ib01, ib02, ib03), (ib10, ib11, ib12, ib13))
    sul = (sul0, sul1)
    cidx = (cidx0, cidx1, cidx2, cidx3)
    asem = (asem0, asem1)
    lsem = (lsem0, lsem1)
    isem = (isem0, isem1)
    rsem = (rsem0, rsem1)
    dsem = (dsem0, dsem1)
    ssem = (ssem0, ssem1)
    clsem = (clsem0, clsem1, clsem2, clsem3)
    csem = (csem0, csem1, csem2, csem3)

    pltpu.sync_copy(ulin_hbm.at[pl.ds(u0, UPW)], ulbuf)
    ul_d = [None, None]
    wg_d = [None, None]

    def ul_load(r, pb):
        ul_d[pb] = pltpu.async_copy(
            ulin_hbm.at[pl.ds(u0 + r * PR, PR)], uidx[pb], lsem[pb])

    def win_gather(r, pb):
        wg_d[pb] = pltpu.async_copy(
            win_hbm.at[uidx[pb]], winu.at[pl.ds(r * PR, PR)], asem[pb])

    ul_load(0, 0)
    ul_load(1, 1)
    for r in range(UPW // PR):
        pb = r & 1
        ul_d[pb].wait()
        if wg_d[pb] is not None:
            wg_d[pb].wait()
        win_gather(r, pb)
        if r + 2 < UPW // PR:
            ul_load(r + 2, pb)
    wg_d[0].wait()
    wg_d[1].wait()

    i_d = [[None] * 4, [None] * 4]
    r_d = [None, None]
    d_d = [None, None]
    s_d = [None, None]

    def idx_load(cc, pb):
        for j in range(4):
            i_d[pb][j] = pltpu.async_copy(
                idx_hbm.at[pl.ds((u0 + cc * CH) * K + j * PR, PR)],
                ib[pb][j], isem[pb])

    def idx_wait(pb):
        for j in range(4):
            i_d[pb][j].wait()

    def rows_start(pb):
        r_d[pb] = [
            pltpu.async_copy(imgp_hbm.at[ib[pb][j]],
                             rows.at[pb, pl.ds(j * PR, PR), :], rsem[pb])
            for j in range(4)]

    def rows_wait(pb):
        for d in r_d[pb]:
            d.wait()

    def dist_load(cc, pb):
        d_d[pb] = pltpu.async_copy(
            dist_hbm.at[pl.ds(wid * NCHUNK + cc, 1), :, :], dbuf.at[pb],
            dsem[pb])

    def scatter_start(pb):
        s_d[pb] = pltpu.async_copy(vals.at[pb], out_hbm.at[sul[pb]],
                                   ssem[pb])

    def compute(cc, pb):
        for g in range(CH // L):
            loff = cc * CH + g * L
            wks = []
            ssum = jnp.zeros((L,), jnp.float32)
            for k in range(K):
                dk = dbuf[pb, 0, k, pl.ds(g * L, L)]
                wk = 1.0 / (dk * dk)
                wks.append(wk)
                ssum = ssum + wk
            rn = 1.0 / ssum
            wns = [wk * rn for wk in wks]
            uglob = iota + (u0 + loff)
            wv = winu[pl.ds(loff, L)]
            ul16 = ulbuf[pl.ds(loff, L)]
            sul[pb][pl.ds(g * L, L)] = jnp.where(wv == uglob, ul16, DUMP)

            def lbody(lane, _):
                ulocal = g * L + lane
                r0 = ulocal * K
                acc0 = jnp.zeros((L,), jnp.float32)
                acc1 = jnp.zeros((L,), jnp.float32)
                lidx = jnp.full((L,), lane, jnp.int32)
                for k in range(K):
                    w = wns[k].at[lidx].get(mode="promise_in_bounds")
                    acc0 = acc0 + rows[pb, r0 + k, pl.ds(0, L)] * w
                    acc1 = acc1 + rows[pb, r0 + k, pl.ds(L, L)] * w
                vals[pb, ulocal, pl.ds(0, L)] = acc0
                vals[pb, ulocal, pl.ds(L, L)] = acc1
                return 0

            lax.fori_loop(0, L, lbody, 0)

    def bgroup(grp, _):
        g0 = grp * GRP
        idx_load(g0, 0)
        idx_load(g0 + 1, 1)
        dist_load(g0, 0)
        dist_load(g0 + 1, 1)
        idx_wait(0)
        rows_start(0)
        for c in range(GRP):
            pb = c & 1
            if c + 1 < GRP:
                idx_wait(pb ^ 1)
                rows_start(pb ^ 1)
            rows_wait(pb)
            if c + 2 < GRP:
                idx_load(g0 + c + 2, pb)
            d_d[pb].wait()
            if s_d[pb] is not None:
                s_d[pb].wait()
            compute(g0 + c, pb)
            scatter_start(pb)
            if c + 2 < GRP:
                dist_load(g0 + c + 2, pb)
        s_d[0].wait()
        s_d[1].wait()
        s_d[0] = None
        s_d[1] = None
        return 0

    lax.fori_loop(0, NCHUNK // GRP, bgroup, 0)

    p0 = wid * HWPW
    cl_d = [None] * 4
    cs_d = [None] * 4

    def copy_load(r, q):
        cl_d[q] = (
            pltpu.async_copy(img_hbm.at[pl.ds(p0 + r * PR, PR), :],
                             crows.at[q], clsem[q]),
            pltpu.async_copy(win_hbm.at[pl.ds(p0 + r * PR, PR)],
                             winc.at[q], clsem[q]))

    def copy_scatter(q):
        cs_d[q] = pltpu.async_copy(crows.at[q], out_hbm.at[cidx[q]],
                                   csem[q])

    def cgroup(grp, _):
        r0g = grp * CGRP
        copy_load(r0g, 0)
        copy_load(r0g + 1, 1)
        for c in range(CGRP):
            q = c & 3
            r = r0g + c
            for d in cl_d[q]:
                d.wait()
            for t in range(PR // L):
                w16 = winc[q, pl.ds(t * L, L)]
                rowid = iota + (p0 + r * PR + t * L)
                cidx[q][pl.ds(t * L, L)] = jnp.where(w16 < 0, rowid, DUMP)
            copy_scatter(q)
            if c + 2 < CGRP:
                qn = (c + 2) & 3
                if cs_d[qn] is not None:
                    cs_d[qn].wait()
                copy_load(r + 2, qn)
        for q in (0, 1, 2, 3):
            if cs_d[q] is not None:
                cs_d[q].wait()
            cs_d[q] = None
        return 0

    lax.fori_loop(0, NCOPY // CGRP, cgroup, 0)


_permute = pl.kernel(
    _permute_body,
    out_type=jax.ShapeDtypeStruct((F_, B), jnp.float32),
    mesh=_mesh,
    compiler_params=_sc_params,
    scratch_types=[
        pltpu.VMEM((PR,), jnp.int32),
        pltpu.VMEM((PR,), jnp.int32),
        pltpu.VMEM((2, PR, B), jnp.float32),
    ] + [pltpu.SemaphoreType.DMA] * 6,
)

_fill = pl.kernel(
    _fill_body,
    out_type=jax.ShapeDtypeStruct((HW + 8, B), jnp.float32),
    mesh=_mesh,
    compiler_params=_sc_params,
    scratch_types=[
        pltpu.VMEM((UPW,), jnp.int32),
        pltpu.VMEM((UPW,), jnp.int32),
        pltpu.VMEM((PR,), jnp.int32),
        pltpu.VMEM((PR,), jnp.int32),
        pltpu.VMEM((PR,), jnp.int32),
        pltpu.VMEM((PR,), jnp.int32),
        pltpu.VMEM((PR,), jnp.int32),
        pltpu.VMEM((PR,), jnp.int32),
        pltpu.VMEM((PR,), jnp.int32),
        pltpu.VMEM((PR,), jnp.int32),
        pltpu.VMEM((PR,), jnp.int32),
        pltpu.VMEM((PR,), jnp.int32),
        pltpu.VMEM((2, CH * K, B), jnp.float32),
        pltpu.VMEM((2, 1, K, CH), jnp.float32),
        pltpu.VMEM((2, CH, B), jnp.float32),
        pltpu.VMEM((CH,), jnp.int32),
        pltpu.VMEM((CH,), jnp.int32),
        pltpu.VMEM((4, PR, B), jnp.float32),
        pltpu.VMEM((4, PR), jnp.int32),
        pltpu.VMEM((PR,), jnp.int32),
        pltpu.VMEM((PR,), jnp.int32),
        pltpu.VMEM((PR,), jnp.int32),
        pltpu.VMEM((PR,), jnp.int32),
    ] + [pltpu.SemaphoreType.DMA] * 20,
)


@jax.jit
def kernel(holed_img, idx, dist, filled_idx, unfilled_idx):
    img_t = jnp.swapaxes(holed_img.reshape(B, HW), 0, 1)
    lin_f = filled_idx[:, 0] * W + filled_idx[:, 1]
    ulin = unfilled_idx[:, 0] * W + unfilled_idx[:, 1]
    img_p = _permute(img_t, lin_f)
    dep = (img_p[0, 0] * 0.0).astype(jnp.int32)
    win = jnp.full((HW,), -1, jnp.int32).at[ulin].max(
        jnp.arange(U, dtype=jnp.int32) + dep)
    dblk = dist.reshape(U // CH, CH, K).swapaxes(1, 2)
    out_t = _fill(img_p, idx.reshape(U * K), dblk, ulin, win, img_t)
    return jnp.swapaxes(out_t[:HW], 0, 1).reshape(B, H, W)


# speedup vs baseline: 1.4332x; 1.0025x over previous
import jax
import jax.numpy as jnp
from jax import lax
from jax.experimental import pallas as pl
from jax.experimental.pallas import tpu as pltpu
from jax.experimental.pallas import tpu_sc as plsc

B, H, W = 32, 512, 512
HW = H * W
U = 131072
F_ = 131072
K = 8
NC, NS, L = 2, 16, 16
NW = NC * NS
UPW = U // NW
CH = 64
NCHUNK = UPW // CH
GRP = 8
FPW = F_ // NW
PR = 128
NPERM = FPW // PR
HWPW = HW // NW
NCOPY = HWPW // PR
CGRP = 8
DUMP = HW
_mesh = plsc.VectorSubcoreMesh(core_axis_name="c", subcore_axis_name="s")
_sc_params = pltpu.CompilerParams(use_tc_tiling_on_sc=False)


def _wid():
    return lax.axis_index("s") * NC + lax.axis_index("c")


def _permute_body(img_hbm, lin_hbm, imgp_hbm, lbuf0, lbuf1, prows,
                  lsem0, lsem1, gsem0, gsem1, psem0, psem1):
    wid = _wid()
    f0 = wid * FPW
    lbuf = (lbuf0, lbuf1)
    lsem = (lsem0, lsem1)
    gsem = (gsem0, gsem1)
    psem = (psem0, psem1)
    lin_d = [None, None]
    g_d = [None, None]
    s_d = [None, None]

    def lin_load(r, pb):
        lin_d[pb] = pltpu.async_copy(
            lin_hbm.at[pl.ds(f0 + r * PR, PR)], lbuf[pb], lsem[pb])

    def gather(pb):
        g_d[pb] = pltpu.async_copy(img_hbm.at[lbuf[pb]], prows.at[pb],
                                   gsem[pb])

    def store(r, pb):
        s_d[pb] = pltpu.async_copy(
            prows.at[pb], imgp_hbm.at[pl.ds(f0 + r * PR, PR), :], psem[pb])

    lin_load(0, 0)
    lin_load(1, 1)
    lin_d[0].wait()
    gather(0)
    for r in range(NPERM):
        pb = r & 1
        if r + 1 < NPERM:
            lin_d[pb ^ 1].wait()
            if s_d[pb ^ 1] is not None:
                s_d[pb ^ 1].wait()
            gather(pb ^ 1)
        g_d[pb].wait()
        store(r, pb)
        if r + 2 < NPERM:
            lin_load(r + 2, pb)
    s_d[0].wait()
    s_d[1].wait()


def _fill_body(imgp_hbm, idx_hbm, dist_hbm, ulin_hbm, win_hbm, img_hbm,
               out_hbm, ulbuf, winu, uidx0, uidx1,
               ib00, ib01, ib02, ib03, ib10, ib11, ib12, ib13,
               rows, dbuf, vals, sul0, sul1, crows, winc,
               cidx0, cidx1, cidx2, cidx3,
               asem0, asem1, lsem0, lsem1, isem0, isem1, rsem0, rsem1,
               dsem0, dsem1, ssem0, ssem1, clsem0, clsem1, clsem2, clsem3,
               csem0, csem1, csem2, csem3):
    wid = _wid()
    u0 = wid * UPW
    iota = lax.broadcasted_iota(jnp.int32, (L,), 0)
    uidx = (uidx0, uidx1)
    ib = ((ib00, ib01, ib02, ib03), (ib10, ib11, ib12, ib13))
    sul = (sul0, sul1)
    cidx = (cidx0, cidx1, cidx2, cidx3)
    asem = (asem0, asem1)
    lsem = (lsem0, lsem1)
    isem = (isem0, isem1)
    rsem = (rsem0, rsem1)
    dsem = (dsem0, dsem1)
    ssem = (ssem0, ssem1)
    clsem = (clsem0, clsem1, clsem2, clsem3)
    csem = (csem0, csem1, csem2, csem3)

    pltpu.sync_copy(ulin_hbm.at[pl.ds(u0, UPW)], ulbuf)
    ul_d = [None, None]
    wg_d = [None, None]

    def ul_load(r, pb):
        ul_d[pb] = pltpu.async_copy(
            ulin_hbm.at[pl.ds(u0 + r * PR, PR)], uidx[pb], lsem[pb])

    def win_gather(r, pb):
        wg_d[pb] = pltpu.async_copy(
            win_hbm.at[uidx[pb]], winu.at[pl.ds(r * PR, PR)], asem[pb])

    ul_load(0, 0)
    ul_load(1, 1)
    for r in range(UPW // PR):
        pb = r & 1
        ul_d[pb].wait()
        if wg_d[pb] is not None:
            wg_d[pb].wait()
        win_gather(r, pb)
        if r + 2 < UPW // PR:
            ul_load(r + 2, pb)
    wg_d[0].wait()
    wg_d[1].wait()

    i_d = [[None] * 4, [None] * 4]
    r_d = [None, None]
    d_d = [None, None]
    s_d = [None, None]

    def idx_load(cc, pb):
        for j in range(4):
            i_d[pb][j] = pltpu.async_copy(
                idx_hbm.at[pl.ds((u0 + cc * CH) * K + j * PR, PR)],
                ib[pb][j], isem[pb])

    def idx_wait(pb):
        for j in range(4):
            i_d[pb][j].wait()

    def rows_start(pb):
        r_d[pb] = [
            pltpu.async_copy(imgp_hbm.at[ib[pb][j]],
                             rows.at[pb, pl.ds(j * PR, PR), :], rsem[pb])
            for j in range(4)]

    def rows_wait(pb):
        for d in r_d[pb]:
            d.wait()

    def dist_load(cc, pb):
        d_d[pb] = pltpu.async_copy(
            dist_hbm.at[pl.ds(wid * NCHUNK + cc, 1), :, :], dbuf.at[pb],
            dsem[pb])

    def scatter_start(pb):
        s_d[pb] = pltpu.async_copy(vals.at[pb], out_hbm.at[sul[pb]],
                                   ssem[pb])

    def compute(cc, pb):
        for g in range(CH // L):
            loff = cc * CH + g * L
            wks = []
            ssum = jnp.zeros((L,), jnp.float32)
            for k in range(K):
                dk = dbuf[pb, 0, k, pl.ds(g * L, L)]
                wk = 1.0 / (dk * dk)
                wks.append(wk)
                ssum = ssum + wk
            rn = 1.0 / ssum
            wns = [wk * rn for wk in wks]
            uglob = iota + (u0 + loff)
            wv = winu[pl.ds(loff, L)]
            ul16 = ulbuf[pl.ds(loff, L)]
            sul[pb][pl.ds(g * L, L)] = jnp.where(wv == uglob, ul16, DUMP)

            def lbody(li, _):
                for dl in range(2):
                    lane = li * 2 + dl
                    ulocal = g * L + lane
                    r0 = ulocal * K
                    lidx = jnp.full((L,), lane, jnp.int32)
                    ws = [wns[k].at[lidx].get(mode="promise_in_bounds")
                          for k in range(K)]
                    for half in range(2):
                        t = [ws[k] * rows[pb, r0 + k, pl.ds(half * L, L)]
                             for k in range(K)]
                        s = ((t[0] + t[1]) + (t[2] + t[3])) +                             ((t[4] + t[5]) + (t[6] + t[7]))
                        vals[pb, ulocal, pl.ds(half * L, L)] = s
                return 0

            lax.fori_loop(0, L // 2, lbody, 0)

    def bgroup(grp, _):
        g0 = grp * GRP
        idx_load(g0, 0)
        idx_load(g0 + 1, 1)
        dist_load(g0, 0)
        dist_load(g0 + 1, 1)
        idx_wait(0)
        rows_start(0)
        for c in range(GRP):
            pb = c & 1
            if c + 1 < GRP:
                idx_wait(pb ^ 1)
                rows_start(pb ^ 1)
            rows_wait(pb)
            if c + 2 < GRP:
                idx_load(g0 + c + 2, pb)
            d_d[pb].wait()
            if s_d[pb] is not None:
                s_d[pb].wait()
            compute(g0 + c, pb)
            scatter_start(pb)
            if c + 2 < GRP:
                dist_load(g0 + c + 2, pb)
        s_d[0].wait()
        s_d[1].wait()
        s_d[0] = None
        s_d[1] = None
        return 0

    lax.fori_loop(0, NCHUNK // GRP, bgroup, 0)

    p0 = wid * HWPW
    cl_d = [None] * 4
    cs_d = [None] * 4

    def copy_load(r, q):
        cl_d[q] = (
            pltpu.async_copy(img_hbm.at[pl.ds(p0 + r * PR, PR), :],
                             crows.at[q], clsem[q]),
            pltpu.async_copy(win_hbm.at[pl.ds(p0 + r * PR, PR)],
                             winc.at[q], clsem[q]))

    def copy_scatter(q):
        cs_d[q] = pltpu.async_copy(crows.at[q], out_hbm.at[cidx[q]],
                                   csem[q])

    def cgroup(grp, _):
        r0g = grp * CGRP
        copy_load(r0g, 0)
        copy_load(r0g + 1, 1)
        for c in range(CGRP):
            q = c & 3
            r = r0g + c
            for d in cl_d[q]:
                d.wait()
            for t in range(PR // L):
                w16 = winc[q, pl.ds(t * L, L)]
                rowid = iota + (p0 + r * PR + t * L)
                cidx[q][pl.ds(t * L, L)] = jnp.where(w16 < 0, rowid, DUMP)
            copy_scatter(q)
            if c + 2 < CGRP:
                qn = (c + 2) & 3
                if cs_d[qn] is not None:
                    cs_d[qn].wait()
                copy_load(r + 2, qn)
        for q in (0, 1, 2, 3):
            if cs_d[q] is not None:
                cs_d[q].wait()
            cs_d[q] = None
        return 0

    lax.fori_loop(0, NCOPY // CGRP, cgroup, 0)


_permute = pl.kernel(
    _permute_body,
    out_type=jax.ShapeDtypeStruct((F_, B), jnp.float32),
    mesh=_mesh,
    compiler_params=_sc_params,
    scratch_types=[
        pltpu.VMEM((PR,), jnp.int32),
        pltpu.VMEM((PR,), jnp.int32),
        pltpu.VMEM((2, PR, B), jnp.float32),
    ] + [pltpu.SemaphoreType.DMA] * 6,
)

_fill = pl.kernel(
    _fill_body,
    out_type=jax.ShapeDtypeStruct((HW + 8, B), jnp.float32),
    mesh=_mesh,
    compiler_params=_sc_params,
    scratch_types=[
        pltpu.VMEM((UPW,), jnp.int32),
        pltpu.VMEM((UPW,), jnp.int32),
        pltpu.VMEM((PR,), jnp.int32),
        pltpu.VMEM((PR,), jnp.int32),
        pltpu.VMEM((PR,), jnp.int32),
        pltpu.VMEM((PR,), jnp.int32),
        pltpu.VMEM((PR,), jnp.int32),
        pltpu.VMEM((PR,), jnp.int32),
        pltpu.VMEM((PR,), jnp.int32),
        pltpu.VMEM((PR,), jnp.int32),
        pltpu.VMEM((PR,), jnp.int32),
        pltpu.VMEM((PR,), jnp.int32),
        pltpu.VMEM((2, CH * K, B), jnp.float32),
        pltpu.VMEM((2, 1, K, CH), jnp.float32),
        pltpu.VMEM((2, CH, B), jnp.float32),
        pltpu.VMEM((CH,), jnp.int32),
        pltpu.VMEM((CH,), jnp.int32),
        pltpu.VMEM((4, PR, B), jnp.float32),
        pltpu.VMEM((4, PR), jnp.int32),
        pltpu.VMEM((PR,), jnp.int32),
        pltpu.VMEM((PR,), jnp.int32),
        pltpu.VMEM((PR,), jnp.int32),
        pltpu.VMEM((PR,), jnp.int32),
    ] + [pltpu.SemaphoreType.DMA] * 20,
)


@jax.jit
def kernel(holed_img, idx, dist, filled_idx, unfilled_idx):
    img_t = jnp.swapaxes(holed_img.reshape(B, HW), 0, 1)
    lin_f = filled_idx[:, 0] * W + filled_idx[:, 1]
    ulin = unfilled_idx[:, 0] * W + unfilled_idx[:, 1]
    img_p = _permute(img_t, lin_f)
    dep = (img_p[0, 0] * 0.0).astype(jnp.int32)
    win = jnp.full((HW,), -1, jnp.int32).at[ulin].max(
        jnp.arange(U, dtype=jnp.int32) + dep)
    dblk = dist.reshape(U // CH, CH, K).swapaxes(1, 2)
    out_t = _fill(img_p, idx.reshape(U * K), dblk, ulin, win, img_t)
    return jnp.swapaxes(out_t[:HW], 0, 1).reshape(B, H, W)
